# Initial kernel scaffold; baseline (speedup 1.0000x reference)
import jax
import jax.numpy as jnp
from jax.experimental import pallas as pl


def kernel(x, edge_index, W1, b1, W2, b2):
    raise NotImplementedError("write your pallas kernel here")


# speedup vs baseline: 29.7711x; 29.7711x over previous
import functools
import jax
import jax.numpy as jnp
from jax import lax
from jax.experimental import pallas as pl
from jax.experimental.pallas import tpu as pltpu
from jax.experimental.pallas import tpu_sc as plsc

_N = 10000
_NP = 10240
_E = 320000
_EP = 327680
_RA = _EP // 128
_NC, _NS = 2, 16
_TR = _RA // (_NC * _NS)
_OUTER, _INNER = 10, 8
_NT = _NP // _NS
_A = 0.1
_K = 10
_C = 16

_mesh = plsc.VectorSubcoreMesh(core_axis_name="c", subcore_axis_name="s")
_sc_params = pltpu.CompilerParams(use_tc_tiling_on_sc=False)


def _zero_agg(zbuf, agg, s):
    def zfill(i, carry):
        zbuf[i] = jnp.zeros((_C,), jnp.float32)
        return carry
    lax.fori_loop(0, _NT, zfill, 0)
    pltpu.sync_copy(zbuf, agg.at[pl.ds(s * _NT, _NT)])


def _sc_deg_body(dstR, out_hbm, didx, ones, zbuf, agg, sem):
    c = lax.axis_index("c")
    s = lax.axis_index("s")
    _zero_agg(zbuf, agg, s)

    def ofill(i, carry):
        ones[i] = jnp.full((_C,), 1.0, jnp.float32)
        return carry
    lax.fori_loop(0, 128, ofill, 0)
    plsc.subcore_barrier()
    base = (c * _NS + s) * _TR
    pltpu.sync_copy(dstR.at[pl.ds(base, _TR)], didx)
    for j in range(_TR):
        pltpu.async_copy(ones, agg.at[didx.at[j]], sem, add=True).wait()
    plsc.subcore_barrier()
    pltpu.sync_copy(agg.at[pl.ds(s * _NT, _NT)],
                    out_hbm.at[c, pl.ds(s * _NT, _NT)])


_sc_deg = pl.kernel(
    _sc_deg_body,
    out_type=jax.ShapeDtypeStruct((_NC, _NP, _C), jnp.float32),
    mesh=_mesh,
    compiler_params=_sc_params,
    scratch_types=[
        pltpu.VMEM((_TR, 128), jnp.int32),
        pltpu.VMEM((128, _C), jnp.float32),
        pltpu.VMEM((_NT, _C), jnp.float32),
        pltpu.VMEM_SHARED((_NP, _C), jnp.float32),
        pltpu.SemaphoreType.DMA,
    ],
)


def _sc_scatter_body(z_hbm, srcR, dstR, out_hbm,
                     sidx, didx, rows, zbuf, agg, gsem, ssem):
    c = lax.axis_index("c")
    s = lax.axis_index("s")
    _zero_agg(zbuf, agg, s)
    plsc.subcore_barrier()
    base = (c * _NS + s) * _TR
    pltpu.sync_copy(srcR.at[pl.ds(base, _TR)], sidx)
    pltpu.sync_copy(dstR.at[pl.ds(base, _TR)], didx)
    for jo in range(_OUTER):
        descs = []
        for ji in range(_INNER):
            j = jo * _INNER + ji
            descs.append(pltpu.async_copy(z_hbm.at[sidx.at[j]],
                                          rows.at[ji], gsem))
        for d in descs:
            d.wait()
        descs = []
        for ji in range(_INNER):
            j = jo * _INNER + ji
            descs.append(pltpu.async_copy(rows.at[ji], agg.at[didx.at[j]],
                                          ssem, add=True))
        for d in descs:
            d.wait()
    plsc.subcore_barrier()
    pltpu.sync_copy(agg.at[pl.ds(s * _NT, _NT)],
                    out_hbm.at[c, pl.ds(s * _NT, _NT)])


_sc_scatter = pl.kernel(
    _sc_scatter_body,
    out_type=jax.ShapeDtypeStruct((_NC, _NP, _C), jnp.float32),
    mesh=_mesh,
    compiler_params=_sc_params,
    scratch_types=[
        pltpu.VMEM((_TR, 128), jnp.int32),
        pltpu.VMEM((_TR, 128), jnp.int32),
        pltpu.VMEM((_INNER, 128, _C), jnp.float32),
        pltpu.VMEM((_NT, _C), jnp.float32),
        pltpu.VMEM_SHARED((_NP, _C), jnp.float32),
        pltpu.SemaphoreType.DMA,
        pltpu.SemaphoreType.DMA,
    ],
)


def _tc_prep_body(x_ref, w1_ref, b1_ref, w2_ref, b2_ref, dega_ref,
                  z0_ref, dinv_ref, dsq_ref):
    xv = x_ref[...]
    h = jnp.dot(xv, w1_ref[...], preferred_element_type=jnp.float32)
    h = jnp.maximum(h + b1_ref[...], 0.0)
    h0 = jnp.dot(h, w2_ref[...], preferred_element_type=jnp.float32)
    h0 = jnp.maximum(h0 + b2_ref[...], 0.0)
    deg = dega_ref[0][:, 0:1] + dega_ref[1][:, 0:1] + 1.0
    rid = lax.broadcasted_iota(jnp.int32, (_NP, 1), 0)
    valid = (rid < _N).astype(jnp.float32)
    dsi = valid * lax.rsqrt(deg)
    z0_ref[...] = dsi * h0
    dinv_ref[...] = jnp.broadcast_to(valid / deg, (_NP, _C))
    dsq_ref[...] = jnp.broadcast_to(valid * jnp.sqrt(deg), (_NP, _C))


_tc_prep = pl.pallas_call(
    _tc_prep_body,
    out_shape=[jax.ShapeDtypeStruct((_NP, _C), jnp.float32)] * 3,
)

_FR = _NP * _C // 128


def _tc_combine_body(z0_ref, dinv_ref, a_ref, z_ref, o_ref):
    u = a_ref[0] + a_ref[1] + z_ref[...]
    o_ref[...] = _A * z0_ref[...] + (1.0 - _A) * dinv_ref[...] * u


_tc_combine = pl.pallas_call(
    _tc_combine_body,
    out_shape=jax.ShapeDtypeStruct((_FR, 128), jnp.float32),
)


def _tc_final_body(z0_ref, dinv_ref, dsq_ref, a_ref, z_ref, o_ref):
    u = a_ref[0] + a_ref[1] + z_ref[...]
    o_ref[...] = dsq_ref[...] * (_A * z0_ref[...]
                                 + (1.0 - _A) * dinv_ref[...] * u)


_tc_final = pl.pallas_call(
    _tc_final_body,
    out_shape=jax.ShapeDtypeStruct((_FR, 128), jnp.float32),
)


def kernel(x, edge_index, W1, b1, W2, b2):
    ei = edge_index.astype(jnp.int32)
    src = jnp.pad(ei[0], (0, _EP - _E),
                  constant_values=_NP - 1).reshape(_RA, 128)
    dst = jnp.pad(ei[1], (0, _EP - _E),
                  constant_values=_NP - 1).reshape(_RA, 128)
    xp = jnp.pad(x, ((0, _NP - _N), (0, 0)))

    dega = _sc_deg(dst)
    z0, dinv, dsq = _tc_prep(xp, W1, b1.reshape(1, -1), W2,
                             b2.reshape(1, -1), dega)
    z0f = z0.reshape(_FR, 128)
    dinvf = dinv.reshape(_FR, 128)
    dsqf = dsq.reshape(_FR, 128)

    z = z0
    for _ in range(_K - 1):
        agg = _sc_scatter(z, src, dst)
        z = _tc_combine(z0f, dinvf, agg.reshape(_NC, _FR, 128),
                        z.reshape(_FR, 128)).reshape(_NP, _C)
    agg = _sc_scatter(z, src, dst)
    out = _tc_final(z0f, dinvf, dsqf, agg.reshape(_NC, _FR, 128),
                    z.reshape(_FR, 128))
    return out.reshape(_NP, _C)[:_N]


# speedup vs baseline: 33.4692x; 1.1242x over previous
import functools
import jax
import jax.numpy as jnp
from jax import lax
from jax.experimental import pallas as pl
from jax.experimental.pallas import tpu as pltpu
from jax.experimental.pallas import tpu_sc as plsc

_N = 10000
_NP = 10240
_E = 320000
_EP = 327680
_RA = _EP // 128
_NC, _NS = 2, 16
_TR = _RA // (_NC * _NS)
_OUTER, _INNER = 10, 8
_NT = _NP // _NS
_A = 0.1
_K = 10
_C = 16

_mesh = plsc.VectorSubcoreMesh(core_axis_name="c", subcore_axis_name="s")
_sc_params = pltpu.CompilerParams(use_tc_tiling_on_sc=False)


def _zero_agg(zbuf, agg, s):
    def zfill(i, carry):
        zbuf[i] = jnp.zeros((_C,), jnp.float32)
        return carry
    lax.fori_loop(0, _NT, zfill, 0)
    pltpu.sync_copy(zbuf, agg.at[pl.ds(s * _NT, _NT)])


def _sc_deg_body(dstR, out_hbm, didx, ones, zbuf, agg, sem):
    c = lax.axis_index("c")
    s = lax.axis_index("s")
    _zero_agg(zbuf, agg, s)

    def ofill(i, carry):
        ones[i] = jnp.full((_C,), 1.0, jnp.float32)
        return carry
    lax.fori_loop(0, 128, ofill, 0)
    plsc.subcore_barrier()
    base = (c * _NS + s) * _TR
    pltpu.sync_copy(dstR.at[pl.ds(base, _TR)], didx)
    for j in range(_TR):
        pltpu.async_copy(ones, agg.at[didx.at[j]], sem, add=True).wait()
    plsc.subcore_barrier()
    pltpu.sync_copy(agg.at[pl.ds(s * _NT, _NT)],
                    out_hbm.at[c, pl.ds(s * _NT, _NT)])


_sc_deg = pl.kernel(
    _sc_deg_body,
    out_type=jax.ShapeDtypeStruct((_NC, _NP, _C), jnp.float32),
    mesh=_mesh,
    compiler_params=_sc_params,
    scratch_types=[
        pltpu.VMEM((_TR, 128), jnp.int32),
        pltpu.VMEM((128, _C), jnp.float32),
        pltpu.VMEM((_NT, _C), jnp.float32),
        pltpu.VMEM_SHARED((_NP, _C), jnp.float32),
        pltpu.SemaphoreType.DMA,
    ],
)


def _sc_scatter_body(z_hbm, srcR, dstR, zeros_hbm, out_hbm,
                     sidx, didx, rows, agg, gsem, ssem):
    c = lax.axis_index("c")
    s = lax.axis_index("s")
    pltpu.sync_copy(zeros_hbm.at[pl.ds(s * _NT, _NT)],
                    agg.at[pl.ds(s * _NT, _NT)])
    plsc.subcore_barrier()
    base = (c * _NS + s) * _TR
    pltpu.sync_copy(srcR.at[pl.ds(base, _TR)], sidx)
    pltpu.sync_copy(dstR.at[pl.ds(base, _TR)], didx)

    def fire_gathers(jo, p):
        return [pltpu.async_copy(z_hbm.at[sidx.at[jo * _INNER + ji]],
                                 rows.at[p, ji], gsem)
                for ji in range(_INNER)]

    def fire_scatters(jo, p):
        return [pltpu.async_copy(rows.at[p, ji],
                                 agg.at[didx.at[jo * _INNER + ji]],
                                 ssem, add=True)
                for ji in range(_INNER)]

    gd = [None] * _OUTER
    sd = [None] * _OUTER
    gd[0] = fire_gathers(0, 0)
    for jo in range(_OUTER):
        p = jo % 2
        if jo + 1 < _OUTER:
            if jo >= 1:
                for d in sd[jo - 1]:
                    d.wait()
            gd[jo + 1] = fire_gathers(jo + 1, 1 - p)
        for d in gd[jo]:
            d.wait()
        sd[jo] = fire_scatters(jo, p)
    for jo in (_OUTER - 2, _OUTER - 1):
        for d in sd[jo]:
            d.wait()
    plsc.subcore_barrier()
    pltpu.sync_copy(agg.at[pl.ds(s * _NT, _NT)],
                    out_hbm.at[c, pl.ds(s * _NT, _NT)])


_sc_scatter = pl.kernel(
    _sc_scatter_body,
    out_type=jax.ShapeDtypeStruct((_NC, _NP, _C), jnp.float32),
    mesh=_mesh,
    compiler_params=_sc_params,
    scratch_types=[
        pltpu.VMEM((_TR, 128), jnp.int32),
        pltpu.VMEM((_TR, 128), jnp.int32),
        pltpu.VMEM((2, _INNER, 128, _C), jnp.float32),
        pltpu.VMEM_SHARED((_NP, _C), jnp.float32),
        pltpu.SemaphoreType.DMA,
        pltpu.SemaphoreType.DMA,
    ],
)


def _tc_prep_body(x_ref, w1_ref, b1_ref, w2_ref, b2_ref, dega_ref,
                  z0_ref, dinv_ref, dsq_ref):
    xv = x_ref[...]
    h = jnp.dot(xv, w1_ref[...], preferred_element_type=jnp.float32)
    h = jnp.maximum(h + b1_ref[...], 0.0)
    h0 = jnp.dot(h, w2_ref[...], preferred_element_type=jnp.float32)
    h0 = jnp.maximum(h0 + b2_ref[...], 0.0)
    deg = dega_ref[0][:, 0:1] + dega_ref[1][:, 0:1] + 1.0
    rid = lax.broadcasted_iota(jnp.int32, (_NP, 1), 0)
    valid = (rid < _N).astype(jnp.float32)
    dsi = valid * lax.rsqrt(deg)
    z0_ref[...] = dsi * h0
    dinv_ref[...] = jnp.broadcast_to(valid / deg, (_NP, _C))
    dsq_ref[...] = jnp.broadcast_to(valid * jnp.sqrt(deg), (_NP, _C))


_tc_prep = pl.pallas_call(
    _tc_prep_body,
    out_shape=[jax.ShapeDtypeStruct((_NP, _C), jnp.float32)] * 3,
)

_FR = _NP * _C // 128


def _tc_combine_body(z0_ref, dinv_ref, a_ref, z_ref, o_ref):
    u = a_ref[0] + a_ref[1] + z_ref[...]
    o_ref[...] = _A * z0_ref[...] + (1.0 - _A) * dinv_ref[...] * u


_tc_combine = pl.pallas_call(
    _tc_combine_body,
    out_shape=jax.ShapeDtypeStruct((_FR, 128), jnp.float32),
)


def _tc_final_body(z0_ref, dinv_ref, dsq_ref, a_ref, z_ref, o_ref):
    u = a_ref[0] + a_ref[1] + z_ref[...]
    o_ref[...] = dsq_ref[...] * (_A * z0_ref[...]
                                 + (1.0 - _A) * dinv_ref[...] * u)


_tc_final = pl.pallas_call(
    _tc_final_body,
    out_shape=jax.ShapeDtypeStruct((_FR, 128), jnp.float32),
)


def kernel(x, edge_index, W1, b1, W2, b2):
    ei = edge_index.astype(jnp.int32)
    src = jnp.pad(ei[0], (0, _EP - _E),
                  constant_values=_NP - 1).reshape(_RA, 128)
    dst = jnp.pad(ei[1], (0, _EP - _E),
                  constant_values=_NP - 1).reshape(_RA, 128)
    xp = jnp.pad(x, ((0, _NP - _N), (0, 0)))

    dega = _sc_deg(dst)
    z0, dinv, dsq = _tc_prep(xp, W1, b1.reshape(1, -1), W2,
                             b2.reshape(1, -1), dega)
    z0f = z0.reshape(_FR, 128)
    dinvf = dinv.reshape(_FR, 128)
    dsqf = dsq.reshape(_FR, 128)

    zeros = jnp.zeros((_NP, _C), jnp.float32)
    z = z0
    for _ in range(_K - 1):
        agg = _sc_scatter(z, src, dst, zeros)
        z = _tc_combine(z0f, dinvf, agg.reshape(_NC, _FR, 128),
                        z.reshape(_FR, 128)).reshape(_NP, _C)
    agg = _sc_scatter(z, src, dst, zeros)
    out = _tc_final(z0f, dinvf, dsqf, agg.reshape(_NC, _FR, 128),
                    z.reshape(_FR, 128))
    return out.reshape(_NP, _C)[:_N]


# speedup vs baseline: 35.4550x; 1.0593x over previous
import functools
import jax
import jax.numpy as jnp
from jax import lax
from jax.experimental import pallas as pl
from jax.experimental.pallas import tpu as pltpu
from jax.experimental.pallas import tpu_sc as plsc

_N = 10000
_NP = 10240
_E = 320000
_EP = 327680
_RA = _EP // 128
_NC, _NS = 2, 16
_TR = _RA // (_NC * _NS)
_OUTER, _INNER = 10, 8
_CH = 1024
_CRA = _EP // _CH
_NT = _NP // _NS
_A = 0.1
_K = 10
_C = 16

_mesh = plsc.VectorSubcoreMesh(core_axis_name="c", subcore_axis_name="s")
_sc_params = pltpu.CompilerParams(use_tc_tiling_on_sc=False)


def _zero_agg(zbuf, agg, s):
    def zfill(i, carry):
        zbuf[i] = jnp.zeros((_C,), jnp.float32)
        return carry
    lax.fori_loop(0, _NT, zfill, 0)
    pltpu.sync_copy(zbuf, agg.at[pl.ds(s * _NT, _NT)])


def _sc_deg_body(dstR, out_hbm, didx, ones, zbuf, agg, sem):
    c = lax.axis_index("c")
    s = lax.axis_index("s")
    _zero_agg(zbuf, agg, s)

    def ofill(i, carry):
        ones[i] = jnp.full((_C,), 1.0, jnp.float32)
        return carry
    lax.fori_loop(0, 128, ofill, 0)
    plsc.subcore_barrier()
    base = (c * _NS + s) * _TR
    pltpu.sync_copy(dstR.at[pl.ds(base, _TR)], didx)
    for j in range(_TR):
        pltpu.async_copy(ones, agg.at[didx.at[j]], sem, add=True).wait()
    plsc.subcore_barrier()
    pltpu.sync_copy(agg.at[pl.ds(s * _NT, _NT)],
                    out_hbm.at[c, pl.ds(s * _NT, _NT)])


_sc_deg = pl.kernel(
    _sc_deg_body,
    out_type=jax.ShapeDtypeStruct((_NC, _NP, _C), jnp.float32),
    mesh=_mesh,
    compiler_params=_sc_params,
    scratch_types=[
        pltpu.VMEM((_TR, 128), jnp.int32),
        pltpu.VMEM((128, _C), jnp.float32),
        pltpu.VMEM((_NT, _C), jnp.float32),
        pltpu.VMEM_SHARED((_NP, _C), jnp.float32),
        pltpu.SemaphoreType.DMA,
    ],
)


def _sc_scatter_body(z_hbm, srcR, dstR, zeros_hbm, out_hbm,
                     sidx, didx, rows, agg, gsem, ssem):
    c = lax.axis_index("c")
    s = lax.axis_index("s")
    pltpu.sync_copy(zeros_hbm.at[pl.ds(s * _NT, _NT)],
                    agg.at[pl.ds(s * _NT, _NT)])
    plsc.subcore_barrier()
    base = (c * _NS + s) * _OUTER
    pltpu.sync_copy(srcR.at[pl.ds(base, _OUTER)], sidx)
    pltpu.sync_copy(dstR.at[pl.ds(base, _OUTER)], didx)

    def fire_gather(jo, p):
        return pltpu.async_copy(z_hbm.at[sidx.at[jo]], rows.at[p], gsem)

    def fire_scatter(jo, p):
        return pltpu.async_copy(rows.at[p], agg.at[didx.at[jo]],
                                ssem, add=True)

    gd = [None] * _OUTER
    sd = [None] * _OUTER
    gd[0] = fire_gather(0, 0)
    for jo in range(_OUTER):
        p = jo % 2
        if jo + 1 < _OUTER:
            if jo >= 1:
                sd[jo - 1].wait()
            gd[jo + 1] = fire_gather(jo + 1, 1 - p)
        gd[jo].wait()
        sd[jo] = fire_scatter(jo, p)
    sd[_OUTER - 2].wait()
    sd[_OUTER - 1].wait()
    plsc.subcore_barrier()
    pltpu.sync_copy(agg.at[pl.ds(s * _NT, _NT)],
                    out_hbm.at[c, pl.ds(s * _NT, _NT)])


_sc_scatter = pl.kernel(
    _sc_scatter_body,
    out_type=jax.ShapeDtypeStruct((_NC, _NP, _C), jnp.float32),
    mesh=_mesh,
    compiler_params=_sc_params,
    scratch_types=[
        pltpu.VMEM((_OUTER, _CH), jnp.int32),
        pltpu.VMEM((_OUTER, _CH), jnp.int32),
        pltpu.VMEM((2, _CH, _C), jnp.float32),
        pltpu.VMEM_SHARED((_NP, _C), jnp.float32),
        pltpu.SemaphoreType.DMA,
        pltpu.SemaphoreType.DMA,
    ],
)


def _tc_prep_body(x_ref, w1_ref, b1_ref, w2_ref, b2_ref, dega_ref,
                  z0_ref, dinv_ref, dsq_ref):
    xv = x_ref[...]
    h = jnp.dot(xv, w1_ref[...], preferred_element_type=jnp.float32)
    h = jnp.maximum(h + b1_ref[...], 0.0)
    h0 = jnp.dot(h, w2_ref[...], preferred_element_type=jnp.float32)
    h0 = jnp.maximum(h0 + b2_ref[...], 0.0)
    deg = dega_ref[0][:, 0:1] + dega_ref[1][:, 0:1] + 1.0
    rid = lax.broadcasted_iota(jnp.int32, (_NP, 1), 0)
    valid = (rid < _N).astype(jnp.float32)
    dsi = valid * lax.rsqrt(deg)
    z0_ref[...] = dsi * h0
    dinv_ref[...] = jnp.broadcast_to(valid / deg, (_NP, _C))
    dsq_ref[...] = jnp.broadcast_to(valid * jnp.sqrt(deg), (_NP, _C))


_tc_prep = pl.pallas_call(
    _tc_prep_body,
    out_shape=[jax.ShapeDtypeStruct((_NP, _C), jnp.float32)] * 3,
)

_FR = _NP * _C // 128


def _tc_combine_body(z0_ref, dinv_ref, a_ref, z_ref, o_ref):
    u = a_ref[0] + a_ref[1] + z_ref[...]
    o_ref[...] = _A * z0_ref[...] + (1.0 - _A) * dinv_ref[...] * u


_tc_combine = pl.pallas_call(
    _tc_combine_body,
    out_shape=jax.ShapeDtypeStruct((_FR, 128), jnp.float32),
)


def _tc_final_body(z0_ref, dinv_ref, dsq_ref, a_ref, z_ref, o_ref):
    u = a_ref[0] + a_ref[1] + z_ref[...]
    o_ref[...] = dsq_ref[...] * (_A * z0_ref[...]
                                 + (1.0 - _A) * dinv_ref[...] * u)


_tc_final = pl.pallas_call(
    _tc_final_body,
    out_shape=jax.ShapeDtypeStruct((_FR, 128), jnp.float32),
)


def kernel(x, edge_index, W1, b1, W2, b2):
    ei = edge_index.astype(jnp.int32)
    src = jnp.pad(ei[0], (0, _EP - _E),
                  constant_values=_NP - 1).reshape(_RA, 128)
    dst = jnp.pad(ei[1], (0, _EP - _E),
                  constant_values=_NP - 1).reshape(_RA, 128)
    xp = jnp.pad(x, ((0, _NP - _N), (0, 0)))

    src1k = src.reshape(_CRA, _CH)
    dst1k = dst.reshape(_CRA, _CH)
    dega = _sc_deg(dst)
    z0, dinv, dsq = _tc_prep(xp, W1, b1.reshape(1, -1), W2,
                             b2.reshape(1, -1), dega)
    z0f = z0.reshape(_FR, 128)
    dinvf = dinv.reshape(_FR, 128)
    dsqf = dsq.reshape(_FR, 128)

    zeros = jnp.zeros((_NP, _C), jnp.float32)
    z = z0
    for _ in range(_K - 1):
        agg = _sc_scatter(z, src1k, dst1k, zeros)
        z = _tc_combine(z0f, dinvf, agg.reshape(_NC, _FR, 128),
                        z.reshape(_FR, 128)).reshape(_NP, _C)
    agg = _sc_scatter(z, src1k, dst1k, zeros)
    out = _tc_final(z0f, dinvf, dsqf, agg.reshape(_NC, _FR, 128),
                    z.reshape(_FR, 128))
    return out.reshape(_NP, _C)[:_N]
